# initial kernel scaffold (unmeasured)
import jax
import jax.numpy as jnp
from jax import lax
from jax.experimental import pallas as pl
from jax.experimental.pallas import tpu as pltpu

N_DEV = 4


def kernel(dy, W):
    m, k = dy.shape
    d, k2 = W.shape
    assert k == k2

    def body(dy_ref, w_ref, out_ref, comm_ref, send_sems, recv_sems):
        my = lax.axis_index("i")
        left = (my - 1) % N_DEV
        right = (my + 1) % N_DEV
        opp = (my + 2) % N_DEV

        barrier_sem = pltpu.get_barrier_semaphore()
        for nbr in (left, right, opp):
            pl.semaphore_signal(
                barrier_sem, inc=1,
                device_id=(nbr,), device_id_type=pl.DeviceIdType.MESH,
            )
        pl.semaphore_wait(barrier_sem, 3)

        partial = lax.dot_general(
            dy_ref[...].astype(jnp.bfloat16),
            w_ref[...].astype(jnp.bfloat16),
            (((1,), (1,)), ((), ())),
            preferred_element_type=jnp.float32,
        )
        comm_ref[0] = partial.astype(jnp.bfloat16)

        sends = []
        for idx, (tgt, slot) in enumerate(((right, 1), (left, 2), (opp, 3))):
            rdma = pltpu.make_async_remote_copy(
                src_ref=comm_ref.at[0],
                dst_ref=comm_ref.at[slot],
                send_sem=send_sems.at[idx],
                recv_sem=recv_sems.at[slot],
                device_id=(tgt,),
                device_id_type=pl.DeviceIdType.MESH,
            )
            rdma.start()
            sends.append((rdma, slot))

        acc = partial
        for rdma, slot in sends:
            rdma.wait_recv()
            acc = acc + comm_ref[slot].astype(jnp.float32)
        out_ref[...] = acc

        for rdma, _ in sends:
            rdma.wait_send()

    return pl.pallas_call(
        body,
        out_shape=jax.ShapeDtypeStruct((m, d), jnp.float32),
        in_specs=[
            pl.BlockSpec(memory_space=pltpu.VMEM),
            pl.BlockSpec(memory_space=pltpu.VMEM),
        ],
        out_specs=pl.BlockSpec(memory_space=pltpu.VMEM),
        scratch_shapes=[
            pltpu.VMEM((N_DEV, m, d), jnp.bfloat16),
            pltpu.SemaphoreType.DMA((3,)),
            pltpu.SemaphoreType.DMA((N_DEV,)),
        ],
        compiler_params=pltpu.CompilerParams(collective_id=0),
    )(dy, W)


# baseline (device time: 44233 ns/iter reference)
import jax
import jax.numpy as jnp
from jax import lax
from jax.experimental import pallas as pl
from jax.experimental.pallas import tpu as pltpu

N_DEV = 4


def kernel(dy, W):
    m, k = dy.shape
    d, k2 = W.shape
    assert k == k2 and m == 1024 and d == 1024

    h = m // 2
    q = m // 4
    e = m // 8
    c = d // 2

    def body(dy_hbm, w_hbm, out_ref, dyv, wv, pacc,
             s256, r256, s128, r128, dsems, wsems, ssems, rsems):
        p = lax.axis_index("i")
        left = (p - 1) % N_DEV
        right = (p + 1) % N_DEV
        my_x = p // 2
        my_y = jnp.bitwise_xor(my_x, p % 2)
        qy = jnp.bitwise_xor(p, 1)
        qx = 3 - p

        offAs = (1 - my_y) * q
        offAk = my_y * q
        offBs = h + (1 - my_x) * q
        offBk = h + my_x * q

        wcp = [
            pltpu.make_async_copy(
                w_hbm.at[pl.ds(ch * c, c), :], wv.at[pl.ds(ch * c, c), :],
                wsems.at[ch])
            for ch in range(2)
        ]
        dy_offs = [offAs, offAk, offBs, offBk]
        dcp = [
            pltpu.make_async_copy(
                dy_hbm.at[pl.ds(dy_offs[i], q), :],
                dyv.at[pl.ds(dy_offs[i], q), :],
                dsems.at[i])
            for i in range(4)
        ]
        for cp in (wcp[0], dcp[0], dcp[1], wcp[1], dcp[2], dcp[3]):
            cp.start()

        barrier_sem = pltpu.get_barrier_semaphore()
        for nbr in (left, right):
            pl.semaphore_signal(
                barrier_sem, inc=1,
                device_id=(nbr,), device_id_type=pl.DeviceIdType.MESH,
            )
        pl.semaphore_wait(barrier_sem, 2)

        nt = (((1,), (1,)), ((), ()))

        def colsl(ch):
            return pl.ds(ch * c, c)

        def dot(row_off, ch):
            return lax.dot_general(
                dyv[pl.ds(row_off, q), :], wv[pl.ds(ch * c, c), :], nt,
                preferred_element_type=jnp.float32)

        def sid(step, chain, ch):
            return (step - 1) * 4 + chain * 2 + ch

        def slot14(step, chain, ch):
            return (0 if step == 1 else 4) + chain * 2 + ch

        def slot23(step, chain, ch):
            return (0 if step == 2 else 4) + chain * 2 + ch

        def start_rdma(sref, rref, sem_id, partner):
            rdma = pltpu.make_async_remote_copy(
                src_ref=sref, dst_ref=rref,
                send_sem=ssems.at[sem_id], recv_sem=rsems.at[sem_id],
                device_id=(partner,), device_id_type=pl.DeviceIdType.MESH,
            )
            rdma.start()
            return rdma

        a1, b1, a2, b2, a3, b3, a4, b4 = ({} for _ in range(8))
        redA, redB = {}, {}

        wcp[0].wait()
        dcp[0].wait()
        for ch in (0, 1):
            if ch == 1:
                wcp[1].wait()
            sl = slot14(1, 0, ch)
            s256[sl] = dot(offAs, ch).astype(jnp.bfloat16)
            a1[ch] = start_rdma(s256.at[sl], r256.at[sl], sid(1, 0, ch), qy)
            if ch == 0:
                dcp[1].wait()
            pacc[pl.ds(offAk, q), colsl(ch)] = dot(offAk, ch)
        dcp[2].wait()
        for ch in (0, 1):
            sl = slot14(1, 1, ch)
            s256[sl] = dot(offBs, ch).astype(jnp.bfloat16)
            b1[ch] = start_rdma(s256.at[sl], r256.at[sl], sid(1, 1, ch), qx)
        dcp[3].wait()
        for ch in (0, 1):
            pacc[pl.ds(offBk, q), colsl(ch)] = dot(offBk, ch)

        for ch in (0, 1):
            a1[ch].wait_recv()
            pacc[pl.ds(offAk, q), colsl(ch)] += (
                r256[slot14(1, 0, ch)].astype(jnp.float32))
            sl = slot23(2, 0, ch)
            s128[sl] = pacc[pl.ds(offAk + (1 - my_x) * e, e),
                            colsl(ch)].astype(jnp.bfloat16)
            a2[ch] = start_rdma(s128.at[sl], r128.at[sl], sid(2, 0, ch), qx)
        for ch in (0, 1):
            b1[ch].wait_recv()
            pacc[pl.ds(offBk, q), colsl(ch)] += (
                r256[slot14(1, 1, ch)].astype(jnp.float32))
            sl = slot23(2, 1, ch)
            s128[sl] = pacc[pl.ds(offBk + (1 - my_y) * e, e),
                            colsl(ch)].astype(jnp.bfloat16)
            b2[ch] = start_rdma(s128.at[sl], r128.at[sl], sid(2, 1, ch), qy)

        for ch in (0, 1):
            a2[ch].wait_recv()
            redA[ch] = (pacc[pl.ds(offAk + my_x * e, e), colsl(ch)]
                        + r128[slot23(2, 0, ch)].astype(jnp.float32))
            out_ref[pl.ds(offAk + my_x * e, e), colsl(ch)] = redA[ch]
            sl = slot23(3, 0, ch)
            s128[sl] = redA[ch].astype(jnp.bfloat16)
            a3[ch] = start_rdma(s128.at[sl], r128.at[sl], sid(3, 0, ch), qx)
        for ch in (0, 1):
            b2[ch].wait_recv()
            redB[ch] = (pacc[pl.ds(offBk + my_y * e, e), colsl(ch)]
                        + r128[slot23(2, 1, ch)].astype(jnp.float32))
            out_ref[pl.ds(offBk + my_y * e, e), colsl(ch)] = redB[ch]
            sl = slot23(3, 1, ch)
            s128[sl] = redB[ch].astype(jnp.bfloat16)
            b3[ch] = start_rdma(s128.at[sl], r128.at[sl], sid(3, 1, ch), qy)

        for ch in (0, 1):
            a3[ch].wait_recv()
            out_ref[pl.ds(offAk + (1 - my_x) * e, e), colsl(ch)] = (
                r128[slot23(3, 0, ch)].astype(jnp.float32))
            sl = slot14(4, 0, ch)
            s256[sl, pl.ds(my_x * e, e)] = redA[ch].astype(jnp.bfloat16)
            s256[sl, pl.ds((1 - my_x) * e, e)] = r128[slot23(3, 0, ch)]
            a4[ch] = start_rdma(s256.at[sl], r256.at[sl], sid(4, 0, ch), qy)
        for ch in (0, 1):
            b3[ch].wait_recv()
            out_ref[pl.ds(offBk + (1 - my_y) * e, e), colsl(ch)] = (
                r128[slot23(3, 1, ch)].astype(jnp.float32))
            sl = slot14(4, 1, ch)
            s256[sl, pl.ds(my_y * e, e)] = redB[ch].astype(jnp.bfloat16)
            s256[sl, pl.ds((1 - my_y) * e, e)] = r128[slot23(3, 1, ch)]
            b4[ch] = start_rdma(s256.at[sl], r256.at[sl], sid(4, 1, ch), qx)

        for ch in (0, 1):
            a4[ch].wait_recv()
            out_ref[pl.ds((1 - my_y) * q, q), colsl(ch)] = (
                r256[slot14(4, 0, ch)].astype(jnp.float32))
        for ch in (0, 1):
            b4[ch].wait_recv()
            out_ref[pl.ds(h + (1 - my_x) * q, q), colsl(ch)] = (
                r256[slot14(4, 1, ch)].astype(jnp.float32))

        for group in (a1, b1, a2, b2, a3, b3, a4, b4):
            for rdma in group.values():
                rdma.wait_send()

    return pl.pallas_call(
        body,
        out_shape=jax.ShapeDtypeStruct((m, d), jnp.float32),
        in_specs=[
            pl.BlockSpec(memory_space=pl.MemorySpace.ANY),
            pl.BlockSpec(memory_space=pl.MemorySpace.ANY),
        ],
        out_specs=pl.BlockSpec(memory_space=pltpu.VMEM),
        scratch_shapes=[
            pltpu.VMEM((m, k), jnp.float32),
            pltpu.VMEM((d, k), jnp.float32),
            pltpu.VMEM((m, d), jnp.float32),
            pltpu.VMEM((8, q, c), jnp.bfloat16),
            pltpu.VMEM((8, q, c), jnp.bfloat16),
            pltpu.VMEM((8, e, c), jnp.bfloat16),
            pltpu.VMEM((8, e, c), jnp.bfloat16),
            pltpu.SemaphoreType.DMA((4,)),
            pltpu.SemaphoreType.DMA((2,)),
            pltpu.SemaphoreType.DMA((16,)),
            pltpu.SemaphoreType.DMA((16,)),
        ],
        compiler_params=pltpu.CompilerParams(
            collective_id=0,
            vmem_limit_bytes=64 * 1024 * 1024,
        ),
    )(dy, W)
